# baseline (speedup 1.0000x reference)
import functools

import jax
import jax.numpy as jnp
from jax import lax
from jax.experimental import pallas as pl
from jax.experimental.pallas import tpu as pltpu
from jax.experimental.pallas import tpu_sc as plsc

_L = 16


@functools.cache
def _make_gather(V, D, R, J):
    info = plsc.get_sparse_core_info()
    NC, NS, L = info.num_cores, info.num_subcores, info.num_lanes
    assert L == _L
    NW = NC * NS
    assert R % NW == 0
    IB = R // NW
    assert IB == 128 and D % 8 == 0
    DH = D // 8
    NIB = R // IB
    DP = D
    NBUF = 2
    mesh = plsc.VectorSubcoreMesh(core_axis_name="c", subcore_axis_name="s")

    @functools.partial(
        pl.kernel,
        mesh=mesh,
        out_type=jax.ShapeDtypeStruct((J, DH, NIB, 8, IB), jnp.float32),
        scratch_types=[
            pltpu.VMEM((IB, J), jnp.int32),
            pltpu.VMEM((J, IB), jnp.int32),
            pltpu.VMEM((NBUF, IB, DP), jnp.float32),
            pltpu.VMEM((NBUF, DH, 1, 8, IB), jnp.float32),
            pltpu.SemaphoreType.DMA((NBUF,)),
            pltpu.SemaphoreType.DMA((NBUF,)),
        ],
        compiler_params=pltpu.CompilerParams(
            use_tc_tiling_on_sc=False, needs_layout_passes=False
        ),
    )
    def k(table_hbm, x_hbm, out_hbm, idx_v, idx_t, rows_v, tile_v, gsem, ssem):
        wid = lax.axis_index("s") * NC + lax.axis_index("c")
        r0 = wid * IB
        pltpu.sync_copy(x_hbm.at[pl.ds(r0, IB)], idx_v)

        lanes = [lax.iota(jnp.int32, _L) + g * _L for g in range(IB // _L)]

        def idx_transpose(j, carry):
            jv = jnp.broadcast_to(j, (_L,))
            for g in range(IB // _L):
                idx_t[j, pl.ds(g * _L, _L)] = plsc.load_gather(idx_v, [lanes[g], jv])
            return carry

        lax.fori_loop(0, J, idx_transpose, 0)

        def gather_start(j, b):
            pltpu.async_copy(
                table_hbm.at[idx_t.at[j]], rows_v.at[b, :, pl.ds(0, D)], gsem.at[b]
            )

        def gather_wait(j, b):
            pltpu.make_async_copy(
                table_hbm.at[idx_t.at[j]], rows_v.at[b, :, pl.ds(0, D)], gsem.at[b]
            ).wait()

        def store_start(j, b):
            pltpu.async_copy(
                tile_v.at[b],
                out_hbm.at[j, pl.ds(0, DH), pl.ds(wid, 1)],
                ssem.at[b],
            )

        def store_wait(j, b):
            pltpu.make_async_copy(
                tile_v.at[b],
                out_hbm.at[j, pl.ds(0, DH), pl.ds(wid, 1)],
                ssem.at[b],
            ).wait()

        for j in range(NBUF):
            gather_start(j, j)

        def body(j, carry):
            b = lax.rem(j, NBUF)
            gather_wait(j, b)

            @pl.when(j >= NBUF)
            def _():
                store_wait(j - NBUF, b)

            for dh in range(DH):
                for dl in range(8):
                    dv = jnp.broadcast_to(jnp.int32(dh * 8 + dl), (_L,))
                    for g in range(IB // _L):
                        tile_v[b, dh, 0, dl, pl.ds(g * _L, _L)] = plsc.load_gather(
                            rows_v.at[b], [lanes[g], dv]
                        )

            store_start(j, b)

            @pl.when(j + NBUF < J)
            def _():
                gather_start(j + NBUF, b)
            return carry

        lax.fori_loop(0, J, body, 0)

        for u in range(NBUF):
            store_wait(J - NBUF + ((u - (J - NBUF)) % NBUF), u)

    return k


def kernel(x, table):
    R, J = x.shape
    V, D = table.shape
    out5 = _make_gather(V, D, R, J)(table, x)
    return out5.transpose(2, 4, 0, 1, 3).reshape(R, J, D)


# speedup vs baseline: 1.1877x; 1.1877x over previous
import functools

import jax
import jax.numpy as jnp
from jax import lax
from jax.experimental import pallas as pl
from jax.experimental.pallas import tpu as pltpu
from jax.experimental.pallas import tpu_sc as plsc

_L = 16


@functools.cache
def _make_gather(V, D, R, J):
    info = plsc.get_sparse_core_info()
    NC, NS, L = info.num_cores, info.num_subcores, info.num_lanes
    assert L == _L
    NW = NC * NS
    assert R % NW == 0
    IB = R // NW
    assert IB == 128 and D % 8 == 0
    DH = D // 8
    NIB = R // IB
    DP = D
    NBUF = 2
    mesh = plsc.VectorSubcoreMesh(core_axis_name="c", subcore_axis_name="s")

    @functools.partial(
        pl.kernel,
        mesh=mesh,
        out_type=jax.ShapeDtypeStruct((J, DH, NIB, 8, IB), jnp.float32),
        scratch_types=[
            pltpu.VMEM((IB, J), jnp.int32),
            pltpu.VMEM((J, IB), jnp.int32),
            pltpu.VMEM((NBUF, IB, DP), jnp.float32),
            pltpu.VMEM((NBUF, DH, 1, 8, IB), jnp.float32),
            pltpu.SemaphoreType.DMA((NBUF,)),
            pltpu.SemaphoreType.DMA((NBUF,)),
        ],
        compiler_params=pltpu.CompilerParams(
            use_tc_tiling_on_sc=False, needs_layout_passes=False
        ),
    )
    def k(table_hbm, x_hbm, out_hbm, idx_v, idx_t, rows_v, tile_v, gsem, ssem):
        wid = lax.axis_index("s") * NC + lax.axis_index("c")
        r0 = wid * IB
        pltpu.sync_copy(x_hbm.at[pl.ds(r0, IB)], idx_v)

        lanes = [lax.iota(jnp.int32, _L) + g * _L for g in range(IB // _L)]

        def idx_transpose(j, carry):
            jv = jnp.broadcast_to(j, (_L,))
            for g in range(IB // _L):
                idx_t[j, pl.ds(g * _L, _L)] = plsc.load_gather(idx_v, [lanes[g], jv])
            return carry

        lax.fori_loop(0, J, idx_transpose, 0)

        def gather_start(j, b):
            pltpu.async_copy(
                table_hbm.at[idx_t.at[j]], rows_v.at[b, :, pl.ds(0, D)], gsem.at[b]
            )

        def gather_wait(j, b):
            pltpu.make_async_copy(
                table_hbm.at[idx_t.at[j]], rows_v.at[b, :, pl.ds(0, D)], gsem.at[b]
            ).wait()

        def store_start(j, b):
            pltpu.async_copy(
                tile_v.at[b],
                out_hbm.at[j, pl.ds(0, DH), pl.ds(wid, 1)],
                ssem.at[b],
            )

        def store_wait(j, b):
            pltpu.make_async_copy(
                tile_v.at[b],
                out_hbm.at[j, pl.ds(0, DH), pl.ds(wid, 1)],
                ssem.at[b],
            ).wait()

        for j in range(NBUF):
            gather_start(j, j)

        def body(j, carry):
            b = lax.rem(j, NBUF)
            gather_wait(j, b)

            @pl.when(j >= NBUF)
            def _():
                store_wait(j - NBUF, b)

            for dh in range(DH):
                for dl in range(8):
                    dv = jnp.broadcast_to(jnp.int32(dh * 8 + dl), (_L,))
                    vals = [
                        plsc.load_gather(rows_v.at[b], [lanes[g], dv])
                        for g in range(IB // _L)
                    ]
                    for g in range(IB // _L):
                        tile_v[b, dh, 0, dl, pl.ds(g * _L, _L)] = vals[g]

            store_start(j, b)

            @pl.when(j + NBUF < J)
            def _():
                gather_start(j + NBUF, b)
            return carry

        lax.fori_loop(0, J, body, 0)

        for u in range(NBUF):
            store_wait(J - NBUF + ((u - (J - NBUF)) % NBUF), u)

    return k


def kernel(x, table):
    R, J = x.shape
    V, D = table.shape
    out5 = _make_gather(V, D, R, J)(table, x)
    return out5.transpose(2, 4, 0, 1, 3).reshape(R, J, D)


# speedup vs baseline: 1.9324x; 1.6270x over previous
import functools

import jax
import jax.numpy as jnp
from jax import lax
from jax.experimental import pallas as pl
from jax.experimental.pallas import tpu as pltpu
from jax.experimental.pallas import tpu_sc as plsc

_L = 16


@functools.cache
def _make_gather(V, D, DP, R, J):
    info = plsc.get_sparse_core_info()
    NC, NS, L = info.num_cores, info.num_subcores, info.num_lanes
    assert L == _L
    NW = NC * NS
    assert R % NW == 0
    IB = R // NW
    assert IB == 128 and D % 8 == 0
    DH = D // 8
    NIB = R // IB
    NBUF = 2
    mesh = plsc.VectorSubcoreMesh(core_axis_name="c", subcore_axis_name="s")

    @functools.partial(
        pl.kernel,
        mesh=mesh,
        out_type=jax.ShapeDtypeStruct((J, DH, NIB, 8, IB), jnp.float32),
        scratch_types=[
            pltpu.VMEM((IB, J), jnp.int32),
            pltpu.VMEM((J, IB), jnp.int32),
            pltpu.VMEM((NBUF, IB, DP), jnp.float32),
            pltpu.VMEM((NBUF, DH, 1, 8, IB), jnp.float32),
            pltpu.SemaphoreType.DMA((NBUF,)),
            pltpu.SemaphoreType.DMA((NBUF,)),
        ],
        compiler_params=pltpu.CompilerParams(
            use_tc_tiling_on_sc=False, needs_layout_passes=False
        ),
    )
    def k(table_hbm, x_hbm, out_hbm, idx_v, idx_t, rows_v, tile_v, gsem, ssem):
        wid = lax.axis_index("s") * NC + lax.axis_index("c")
        r0 = wid * IB
        pltpu.sync_copy(x_hbm.at[pl.ds(r0, IB)], idx_v)

        lanes = [lax.iota(jnp.int32, _L) + g * _L for g in range(IB // _L)]

        def idx_transpose(j, carry):
            jv = jnp.broadcast_to(j, (_L,))
            for g in range(IB // _L):
                idx_t[j, pl.ds(g * _L, _L)] = plsc.load_gather(idx_v, [lanes[g], jv])
            return carry

        lax.fori_loop(0, J, idx_transpose, 0)

        def gather_start(j, b):
            pltpu.async_copy(
                table_hbm.at[idx_t.at[j]], rows_v.at[b], gsem.at[b]
            )

        def gather_wait(j, b):
            pltpu.make_async_copy(
                table_hbm.at[idx_t.at[j]], rows_v.at[b], gsem.at[b]
            ).wait()

        def store_start(j, b):
            pltpu.async_copy(
                tile_v.at[b],
                out_hbm.at[j, pl.ds(0, DH), pl.ds(wid, 1)],
                ssem.at[b],
            )

        def store_wait(j, b):
            pltpu.make_async_copy(
                tile_v.at[b],
                out_hbm.at[j, pl.ds(0, DH), pl.ds(wid, 1)],
                ssem.at[b],
            ).wait()

        for j in range(NBUF):
            gather_start(j, j)

        def body(j, carry):
            b = lax.rem(j, NBUF)
            gather_wait(j, b)

            @pl.when(j >= NBUF)
            def _():
                store_wait(j - NBUF, b)

            for dh in range(DH):
                for dl in range(8):
                    dv = jnp.broadcast_to(jnp.int32(dh * 8 + dl), (_L,))
                    vals = [
                        plsc.load_gather(rows_v.at[b], [lanes[g], dv])
                        for g in range(IB // _L)
                    ]
                    for g in range(IB // _L):
                        tile_v[b, dh, 0, dl, pl.ds(g * _L, _L)] = vals[g]

            store_start(j, b)

            @pl.when(j + NBUF < J)
            def _():
                gather_start(j + NBUF, b)
            return carry

        lax.fori_loop(0, J, body, 0)

        for u in range(NBUF):
            store_wait(J - NBUF + ((u - (J - NBUF)) % NBUF), u)

    return k


def kernel(x, table):
    R, J = x.shape
    V, D = table.shape
    DP = D + 8
    table_p = jnp.pad(table, ((0, 0), (0, DP - D)))
    out5 = _make_gather(V, D, DP, R, J)(table_p, x)
    return out5.transpose(2, 4, 0, 1, 3).reshape(R, J, D)


# speedup vs baseline: 2.5287x; 1.3086x over previous
import functools

import jax
import jax.numpy as jnp
from jax import lax
from jax.experimental import pallas as pl
from jax.experimental.pallas import tpu as pltpu
from jax.experimental.pallas import tpu_sc as plsc

_L = 16


@functools.cache
def _make_gather(V, D, R, J):
    info = plsc.get_sparse_core_info()
    NC, NS, L = info.num_cores, info.num_subcores, info.num_lanes
    assert L == _L
    NW = NC * NS
    assert R % NW == 0
    IB = R // NW
    assert IB == 128 and D % 8 == 0
    DH = D // 8
    NIB = R // IB
    NBUF = 2
    mesh = plsc.VectorSubcoreMesh(core_axis_name="c", subcore_axis_name="s")

    @functools.partial(
        pl.kernel,
        mesh=mesh,
        out_type=jax.ShapeDtypeStruct((J, DH, NIB, 8, IB), jnp.float32),
        scratch_types=[
            pltpu.VMEM((IB, J), jnp.int32),
            pltpu.VMEM((J, IB), jnp.int32),
            pltpu.VMEM((NBUF, IB, D), jnp.float32),
            pltpu.VMEM((NBUF, DH, 1, 8, IB + 8), jnp.float32),
            pltpu.SemaphoreType.DMA((NBUF,)),
            pltpu.SemaphoreType.DMA((NBUF,)),
        ],
        compiler_params=pltpu.CompilerParams(
            use_tc_tiling_on_sc=False, needs_layout_passes=False
        ),
    )
    def k(table_hbm, x_hbm, out_hbm, idx_v, idx_t, rows_v, tile_v, gsem, ssem):
        wid = lax.axis_index("s") * NC + lax.axis_index("c")
        r0 = wid * IB
        pltpu.sync_copy(x_hbm.at[pl.ds(r0, IB)], idx_v)

        lanes = [lax.iota(jnp.int32, _L) + g * _L for g in range(IB // _L)]
        dh_of = [(lanes[0] >> 3) + dg * 2 for dg in range(D // _L)]
        dl_of = lanes[0] & 7
        zero_v = jnp.broadcast_to(jnp.int32(0), (_L,))

        def idx_transpose(j, carry):
            jv = jnp.broadcast_to(j, (_L,))
            for g in range(IB // _L):
                idx_t[j, pl.ds(g * _L, _L)] = plsc.load_gather(idx_v, [lanes[g], jv])
            return carry

        lax.fori_loop(0, J, idx_transpose, 0)

        def gather_start(j, b):
            pltpu.async_copy(
                table_hbm.at[idx_t.at[j]], rows_v.at[b], gsem.at[b]
            )

        def gather_wait(j, b):
            pltpu.make_async_copy(
                table_hbm.at[idx_t.at[j]], rows_v.at[b], gsem.at[b]
            ).wait()

        def store_start(j, b):
            pltpu.async_copy(
                tile_v.at[b, :, :, :, pl.ds(0, IB)],
                out_hbm.at[j, pl.ds(0, DH), pl.ds(wid, 1)],
                ssem.at[b],
            )

        def store_wait(j, b):
            pltpu.make_async_copy(
                tile_v.at[b, :, :, :, pl.ds(0, IB)],
                out_hbm.at[j, pl.ds(0, DH), pl.ds(wid, 1)],
                ssem.at[b],
            ).wait()

        for j in range(NBUF):
            gather_start(j, j)

        def body(j, carry):
            b = lax.rem(j, NBUF)
            gather_wait(j, b)

            @pl.when(j >= NBUF)
            def _():
                store_wait(j - NBUF, b)

            for i in range(IB):
                iv = jnp.broadcast_to(jnp.int32(i), (_L,))
                vals = [
                    rows_v[b, i, pl.ds(dg * _L, _L)] for dg in range(D // _L)
                ]
                for dg in range(D // _L):
                    plsc.store_scatter(
                        tile_v.at[b], [dh_of[dg], zero_v, dl_of, iv], vals[dg]
                    )

            store_start(j, b)

            @pl.when(j + NBUF < J)
            def _():
                gather_start(j + NBUF, b)
            return carry

        lax.fori_loop(0, J, body, 0)

        for u in range(NBUF):
            store_wait(J - NBUF + ((u - (J - NBUF)) % NBUF), u)

    return k


def kernel(x, table):
    R, J = x.shape
    V, D = table.shape
    out5 = _make_gather(V, D, R, J)(table, x)
    return out5.transpose(2, 4, 0, 1, 3).reshape(R, J, D)
